# baseline (speedup 1.0000x reference)
import functools

import jax
import jax.numpy as jnp
from jax import lax
from jax.experimental import pallas as pl
from jax.experimental.pallas import tpu as pltpu
from jax.experimental.pallas import tpu_sc as plsc

_EPS = 1e-5


def _gn(x, g, b):
    m = jnp.mean(x, axis=-1, keepdims=True)
    v = jnp.mean((x - m) ** 2, axis=-1, keepdims=True)
    return (x - m) * jax.lax.rsqrt(v + _EPS) * g + b


def _node_pre_body(agts_ref, ctx_ref, actr_ref, cctr_ref, qWT_ref, qg_ref,
                   qb_ref, WqT_ref, WcT_ref, aWT_ref, d0WT_ref,
                   th_ref, tw_ref, a0_ref):
    agts = agts_ref[...]
    ctx = ctx_ref[...]
    q = jax.nn.relu(_gn(jnp.dot(agts, qWT_ref[...],
                                preferred_element_type=jnp.float32),
                        qg_ref[...], qb_ref[...]))

    def pack(feat, proj):
        fb = jax.lax.bitcast_convert_type(
            feat.astype(jnp.bfloat16).astype(jnp.float32), jnp.uint32)
        pb = jax.lax.bitcast_convert_type(
            proj.astype(jnp.bfloat16).astype(jnp.float32), jnp.uint32)
        return jax.lax.bitcast_convert_type(
            pb | jax.lax.shift_right_logical(fb, jnp.uint32(16)), jnp.int32)

    th_ref[...] = pack(
        jnp.dot(q, WqT_ref[...], preferred_element_type=jnp.float32),
        jnp.dot(actr_ref[...], d0WT_ref[...],
                preferred_element_type=jnp.float32))
    tw_ref[...] = pack(
        jnp.dot(ctx, WcT_ref[...], preferred_element_type=jnp.float32),
        jnp.dot(cctr_ref[...], d0WT_ref[...],
                preferred_element_type=jnp.float32))
    a0_ref[...] = jnp.dot(agts, aWT_ref[...], preferred_element_type=jnp.float32)


def _edge_body(gh_ref, gw_ref, d0b_ref, d1WT_ref, d1g_ref, d1b_ref,
               WdT_ref, c0g_ref, c0b_ref, c1WT_ref, msg_ref):
    gh = gh_ref[...]
    gw = gw_ref[...]
    _f32 = lambda x: jax.lax.bitcast_convert_type(x, jnp.float32)
    feat = lambda g: _f32(jax.lax.shift_left(g, 16))
    proj = lambda g: _f32(g & jnp.int32(-65536))
    bf = lambda x: x.astype(jnp.bfloat16)
    d0 = jax.nn.relu(proj(gh) - proj(gw) + d0b_ref[...])
    d1 = jax.nn.relu(_gn(jnp.dot(bf(d0), d1WT_ref[...],
                                 preferred_element_type=jnp.float32),
                         d1g_ref[...], d1b_ref[...]))
    pre = (jnp.dot(bf(d1), WdT_ref[...], preferred_element_type=jnp.float32)
           + feat(gh) + feat(gw))
    h = jax.nn.relu(_gn(pre, c0g_ref[...], c0b_ref[...]))
    msg_ref[...] = jnp.dot(bf(h), c1WT_ref[...],
                           preferred_element_type=jnp.float32)


def _remap_body(hi_ref, hi0_ref, hi1_ref, *, n2):
    v = hi_ref[...]
    hi0_ref[...] = jnp.where(v < n2, v, n2)
    hi1_ref[...] = jnp.where(v >= n2, v - n2, n2)


def _final_body(a0_ref, p_ref, res_ref, ng_ref, nb_ref, lWT_ref,
                lg_ref, lb_ref, out_ref):
    a = a0_ref[...] + p_ref[...]
    a = jax.nn.relu(_gn(a, ng_ref[...], nb_ref[...]))
    a = _gn(jnp.dot(a, lWT_ref[...], preferred_element_type=jnp.float32),
            lg_ref[...], lb_ref[...])
    out_ref[...] = jax.nn.relu(a + res_ref[...])


def _make_gather(N, E, D):
    info = plsc.get_sparse_core_info()
    NC, NS = info.num_cores, info.num_subcores
    NW = NC * NS
    EP = E // NW
    CE = 40
    ITERS = EP // CE
    assert EP % CE == 0 and E % NW == 0

    assert ITERS % 2 == 0
    mesh = plsc.VectorSubcoreMesh(core_axis_name="c", subcore_axis_name="s")
    bshape = jax.ShapeDtypeStruct((E, D), jnp.int32)

    @functools.partial(
        pl.kernel, mesh=mesh,
        out_type=[bshape, bshape],
        scratch_types=[
            pltpu.VMEM((EP,), jnp.int32),
            pltpu.VMEM((EP,), jnp.int32),
            pltpu.VMEM((CE, D), jnp.int32),
            pltpu.VMEM((CE, D), jnp.int32),
            pltpu.VMEM((CE, D), jnp.int32),
            pltpu.VMEM((CE, D), jnp.int32),
            pltpu.SemaphoreType.DMA,
            pltpu.SemaphoreType.DMA,
            pltpu.SemaphoreType.DMA,
            pltpu.SemaphoreType.DMA,
        ],
    )
    def gather(th_hbm, tw_hbm, hi_hbm, wi_hbm, oh_hbm, ow_hbm,
               idxh_v, idxw_v, rhA, rwA, rhB, rwB, sHA, sWA, sHB, sWB):
        wid = lax.axis_index("s") * NC + lax.axis_index("c")
        base0 = wid * EP

        pltpu.sync_copy(hi_hbm.at[pl.ds(base0, EP)], idxh_v)
        pltpu.sync_copy(wi_hbm.at[pl.ds(base0, EP)], idxw_v)

        def start_gA(loc):
            pltpu.async_copy(th_hbm.at[idxh_v.at[pl.ds(loc, CE)]], rhA, sHA)
            pltpu.async_copy(tw_hbm.at[idxw_v.at[pl.ds(loc, CE)]], rwA, sWA)

        def start_gB(loc):
            pltpu.async_copy(th_hbm.at[idxh_v.at[pl.ds(loc, CE)]], rhB, sHB)
            pltpu.async_copy(tw_hbm.at[idxw_v.at[pl.ds(loc, CE)]], rwB, sWB)

        start_gA(0)
        start_gB(CE)

        def body(k, carry):
            locA = pl.multiple_of(2 * k * CE, 8)
            locB = locA + CE
            pltpu.make_async_copy(th_hbm.at[idxh_v.at[pl.ds(locA, CE)]],
                                  rhA, sHA).wait()
            pltpu.make_async_copy(tw_hbm.at[idxw_v.at[pl.ds(locA, CE)]],
                                  rwA, sWA).wait()
            w1 = pltpu.async_copy(rhA, oh_hbm.at[pl.ds(base0 + locA, CE)], sHA)
            w2 = pltpu.async_copy(rwA, ow_hbm.at[pl.ds(base0 + locA, CE)], sWA)
            pltpu.make_async_copy(th_hbm.at[idxh_v.at[pl.ds(locB, CE)]],
                                  rhB, sHB).wait()
            pltpu.make_async_copy(tw_hbm.at[idxw_v.at[pl.ds(locB, CE)]],
                                  rwB, sWB).wait()
            w3 = pltpu.async_copy(rhB, oh_hbm.at[pl.ds(base0 + locB, CE)], sHB)
            w4 = pltpu.async_copy(rwB, ow_hbm.at[pl.ds(base0 + locB, CE)], sWB)
            w1.wait()
            w2.wait()

            @pl.when(locA + 2 * CE < EP)
            def _():
                start_gA(pl.multiple_of(locA + 2 * CE, 8))

            w3.wait()
            w4.wait()

            @pl.when(locB + 2 * CE < EP)
            def _():
                start_gB(pl.multiple_of(locB + 2 * CE, 8))

            return carry

        lax.fori_loop(0, ITERS // 2, body, 0)

    return gather


def _make_scatter(N, E, D):
    info = plsc.get_sparse_core_info()
    NC, NS = info.num_cores, info.num_subcores
    assert NC == 2
    N2 = N // NC
    NR = N2 + 8
    EP = E // NS
    CS = 200
    ITERS = EP // CS
    RT = (NR // NS) // 8 * 8
    TAIL = NR - NS * RT
    assert EP % CS == 0 and CS % 8 == 0 and TAIL % 8 == 0 and N % (2 * 8) == 0

    mesh = plsc.VectorSubcoreMesh(core_axis_name="c", subcore_axis_name="s")

    assert ITERS % 2 == 0

    @functools.partial(
        pl.kernel, mesh=mesh,
        out_type=jax.ShapeDtypeStruct((NC * NR, D), jnp.float32),
        scratch_types=[
            pltpu.VMEM((CS,), jnp.int32),
            pltpu.VMEM((CS,), jnp.int32),
            pltpu.VMEM((CS, D), jnp.float32),
            pltpu.VMEM((CS, D), jnp.float32),
            pltpu.VMEM_SHARED((NR, D), jnp.float32),
            pltpu.SemaphoreType.DMA,
            pltpu.SemaphoreType.DMA,
            pltpu.SemaphoreType.DMA,
            pltpu.SemaphoreType.DMA,
        ],
    )
    def scatter(msg_hbm, hic_hbm, zeros_hbm, out_hbm,
                idxA, idxB, bufA, bufB, acc_sh, sIA, sIB, sMA, sMB):
        cid = lax.axis_index("c")
        sid = lax.axis_index("s")
        base0 = sid * EP

        def start_A(loc):
            pltpu.async_copy(hic_hbm.at[pl.ds(cid * E + loc, CS)], idxA, sIA)
            pltpu.async_copy(msg_hbm.at[pl.ds(loc, CS)], bufA, sMA)

        def start_B(loc):
            pltpu.async_copy(hic_hbm.at[pl.ds(cid * E + loc, CS)], idxB, sIB)
            pltpu.async_copy(msg_hbm.at[pl.ds(loc, CS)], bufB, sMB)

        start_A(pl.multiple_of(base0, 8))
        start_B(pl.multiple_of(base0 + CS, 8))

        pltpu.sync_copy(zeros_hbm.at[pl.ds(sid * RT, RT)],
                        acc_sh.at[pl.ds(sid * RT, RT)])
        if TAIL:
            @pl.when(sid == 0)
            def _():
                pltpu.sync_copy(zeros_hbm.at[pl.ds(NS * RT, TAIL)],
                                acc_sh.at[pl.ds(NS * RT, TAIL)])
        plsc.subcore_barrier()

        def body(k, carry):
            locA = pl.multiple_of(base0 + 2 * k * CS, 8)
            locB = locA + CS
            pltpu.make_async_copy(hic_hbm.at[pl.ds(locA, CS)], idxA, sIA).wait()
            pltpu.make_async_copy(msg_hbm.at[pl.ds(locA, CS)], bufA, sMA).wait()
            pltpu.sync_copy(bufA, acc_sh.at[idxA], add=True)

            @pl.when(2 * k + 2 < ITERS)
            def _():
                start_A(pl.multiple_of(locA + 2 * CS, 8))

            pltpu.make_async_copy(hic_hbm.at[pl.ds(locB, CS)], idxB, sIB).wait()
            pltpu.make_async_copy(msg_hbm.at[pl.ds(locB, CS)], bufB, sMB).wait()
            pltpu.sync_copy(bufB, acc_sh.at[idxB], add=True)

            @pl.when(2 * k + 3 < ITERS)
            def _():
                start_B(pl.multiple_of(locB + 2 * CS, 8))

            return carry

        lax.fori_loop(0, ITERS // 2, body, 0)
        plsc.subcore_barrier()
        pltpu.sync_copy(acc_sh.at[pl.ds(sid * RT, RT)],
                        out_hbm.at[pl.ds(cid * NR + sid * RT, RT)])
        if TAIL:
            @pl.when(sid == 0)
            def _():
                pltpu.sync_copy(acc_sh.at[pl.ds(NS * RT, TAIL)],
                                out_hbm.at[pl.ds(cid * NR + NS * RT, TAIL)])

    return scatter


def kernel(agts, agt_ctrs, ctx, ctx_ctrs, hi, wi,
           dist0_W, dist0_b, dist1_W, dist1_g, dist1_b,
           query_W, query_g, query_b,
           ctx0_W, ctx0_g, ctx0_b, ctx1_W,
           agt_W, norm_g, norm_b, lin_W, lin_g, lin_b):
    N, D = agts.shape
    E = hi.shape[0]

    r2 = lambda v: v.reshape(1, D)
    hi = hi.astype(jnp.int32)
    wi = wi.astype(jnp.int32)

    BN = 2000
    grid_n = N // BN
    row_spec = pl.BlockSpec((BN, D), lambda i: (i, 0))
    tab_spec = pl.BlockSpec((BN, D), lambda i: (i, 0))
    ctr_spec = pl.BlockSpec((BN, 2), lambda i: (i, 0))
    full = lambda s: pl.BlockSpec(s, lambda i: tuple(0 for _ in s))
    table_h, table_w, a0 = pl.pallas_call(
        _node_pre_body,
        grid=(grid_n,),
        in_specs=[row_spec, row_spec, ctr_spec, ctr_spec, full((D, D)),
                  full((1, D)), full((1, D)), full((D, D)), full((D, D)),
                  full((D, D)), full((2, D))],
        out_specs=[tab_spec, tab_spec, row_spec],
        out_shape=[jax.ShapeDtypeStruct((N, D), jnp.int32),
                   jax.ShapeDtypeStruct((N, D), jnp.int32),
                   jax.ShapeDtypeStruct((N, D), jnp.float32)],
    )(agts, ctx, agt_ctrs, ctx_ctrs, query_W.T, r2(query_g), r2(query_b),
      ctx0_W[:, D:2 * D].T, ctx0_W[:, 2 * D:].T, agt_W.T, dist0_W.T)

    gh, gw = _make_gather(N, E, D)(table_h, table_w, hi, wi)

    BE = 2000
    grid_e = E // BE
    espec = pl.BlockSpec((BE, D), lambda i: (i, 0))
    mspec = pl.BlockSpec((BE, D), lambda i: (i, 0))
    msg = pl.pallas_call(
        _edge_body,
        grid=(grid_e,),
        in_specs=[espec, espec, full((1, D)), full((D, D)),
                  full((1, D)), full((1, D)), full((D, D)), full((1, D)),
                  full((1, D)), full((D, D))],
        out_specs=mspec,
        out_shape=jax.ShapeDtypeStruct((E, D), jnp.float32),
    )(gh, gw, r2(dist0_b), dist1_W.T.astype(jnp.bfloat16), r2(dist1_g),
      r2(dist1_b), ctx0_W[:, :D].T.astype(jnp.bfloat16), r2(ctx0_g),
      r2(ctx0_b), ctx1_W.T.astype(jnp.bfloat16))

    hi2d = hi.reshape(E // D, D)
    ispec = pl.BlockSpec((E // D, D), lambda: (0, 0))
    hi0, hi1 = pl.pallas_call(
        functools.partial(_remap_body, n2=N // 2),
        in_specs=[ispec],
        out_specs=[ispec, ispec],
        out_shape=[jax.ShapeDtypeStruct((E // D, D), jnp.int32)] * 2,
    )(hi2d)
    hic = jnp.concatenate([hi0.reshape(E), hi1.reshape(E)])

    NR = N // 2 + 8
    zeros = jnp.zeros((NR, D), jnp.float32)
    parts = _make_scatter(N, E, D)(msg, hic, zeros)
    p = jnp.concatenate([parts[:N // 2], parts[NR:NR + N // 2]], axis=0)

    out = pl.pallas_call(
        _final_body,
        grid=(grid_n,),
        in_specs=[row_spec, row_spec, row_spec, full((1, D)),
                  full((1, D)), full((D, D)), full((1, D)), full((1, D))],
        out_specs=row_spec,
        out_shape=jax.ShapeDtypeStruct((N, D), jnp.float32),
    )(a0, p, agts, r2(norm_g), r2(norm_b), lin_W.T, r2(lin_g), r2(lin_b))
    return out


# speedup vs baseline: 1.0308x; 1.0308x over previous
import functools

import jax
import jax.numpy as jnp
from jax import lax
from jax.experimental import pallas as pl
from jax.experimental.pallas import tpu as pltpu
from jax.experimental.pallas import tpu_sc as plsc

_EPS = 1e-5


def _gn(x, g, b):
    m = jnp.mean(x, axis=-1, keepdims=True)
    v = jnp.mean((x - m) ** 2, axis=-1, keepdims=True)
    return (x - m) * jax.lax.rsqrt(v + _EPS) * g + b


def _node_pre_body(agts_ref, ctx_ref, actr_ref, cctr_ref, qWT_ref, qg_ref,
                   qb_ref, WqT_ref, WcT_ref, aWT_ref, d0WT_ref,
                   th_ref, tw_ref, a0_ref):
    agts = agts_ref[...]
    ctx = ctx_ref[...]
    q = jax.nn.relu(_gn(jnp.dot(agts, qWT_ref[...],
                                preferred_element_type=jnp.float32),
                        qg_ref[...], qb_ref[...]))

    def pack(feat, proj):
        fb = jax.lax.bitcast_convert_type(
            feat.astype(jnp.bfloat16).astype(jnp.float32), jnp.uint32)
        pb = jax.lax.bitcast_convert_type(
            proj.astype(jnp.bfloat16).astype(jnp.float32), jnp.uint32)
        return jax.lax.bitcast_convert_type(
            pb | jax.lax.shift_right_logical(fb, jnp.uint32(16)), jnp.int32)

    th_ref[...] = pack(
        jnp.dot(q, WqT_ref[...], preferred_element_type=jnp.float32),
        jnp.dot(actr_ref[...], d0WT_ref[...],
                preferred_element_type=jnp.float32))
    tw_ref[...] = pack(
        jnp.dot(ctx, WcT_ref[...], preferred_element_type=jnp.float32),
        jnp.dot(cctr_ref[...], d0WT_ref[...],
                preferred_element_type=jnp.float32))
    a0_ref[...] = jnp.dot(agts, aWT_ref[...], preferred_element_type=jnp.float32)


def _edge_body(gh_ref, gw_ref, d0b_ref, d1WT_ref, d1g_ref, d1b_ref,
               WdT_ref, c0g_ref, c0b_ref, c1WT_ref, msg_ref):
    gh = gh_ref[...]
    gw = gw_ref[...]
    _f32 = lambda x: jax.lax.bitcast_convert_type(x, jnp.float32)
    feat = lambda g: _f32(jax.lax.shift_left(g, 16))
    proj = lambda g: _f32(g & jnp.int32(-65536))
    bf = lambda x: x.astype(jnp.bfloat16)
    d0 = jax.nn.relu(proj(gh) - proj(gw) + d0b_ref[...])
    d1 = jax.nn.relu(_gn(jnp.dot(bf(d0), d1WT_ref[...],
                                 preferred_element_type=jnp.float32),
                         d1g_ref[...], d1b_ref[...]))
    pre = (jnp.dot(bf(d1), WdT_ref[...], preferred_element_type=jnp.float32)
           + feat(gh) + feat(gw))
    h = jax.nn.relu(_gn(pre, c0g_ref[...], c0b_ref[...]))
    msg_ref[...] = jnp.dot(bf(h), c1WT_ref[...],
                           preferred_element_type=jnp.float32)


def _remap_body(hi_ref, hi0_ref, hi1_ref, *, n2):
    v = hi_ref[...]
    hi0_ref[...] = jnp.where(v < n2, v, n2)
    hi1_ref[...] = jnp.where(v >= n2, v - n2, n2)


def _final_body(a0_ref, p_ref, res_ref, ng_ref, nb_ref, lWT_ref,
                lg_ref, lb_ref, out_ref):
    a = a0_ref[...] + p_ref[...]
    a = jax.nn.relu(_gn(a, ng_ref[...], nb_ref[...]))
    a = _gn(jnp.dot(a, lWT_ref[...], preferred_element_type=jnp.float32),
            lg_ref[...], lb_ref[...])
    out_ref[...] = jax.nn.relu(a + res_ref[...])


def _make_gather(N, E, D):
    info = plsc.get_sparse_core_info()
    NC, NS = info.num_cores, info.num_subcores
    NW = NC * NS
    EP = E // NW
    CE = 200
    ITERS = EP // CE
    assert EP % CE == 0 and E % NW == 0

    assert ITERS % 2 == 0
    mesh = plsc.VectorSubcoreMesh(core_axis_name="c", subcore_axis_name="s")
    bshape = jax.ShapeDtypeStruct((E, D), jnp.int32)

    @functools.partial(
        pl.kernel, mesh=mesh,
        out_type=[bshape, bshape],
        scratch_types=[
            pltpu.VMEM((EP,), jnp.int32),
            pltpu.VMEM((EP,), jnp.int32),
            pltpu.VMEM((CE, D), jnp.int32),
            pltpu.VMEM((CE, D), jnp.int32),
            pltpu.VMEM((CE, D), jnp.int32),
            pltpu.VMEM((CE, D), jnp.int32),
            pltpu.SemaphoreType.DMA,
            pltpu.SemaphoreType.DMA,
            pltpu.SemaphoreType.DMA,
            pltpu.SemaphoreType.DMA,
        ],
    )
    def gather(th_hbm, tw_hbm, hi_hbm, wi_hbm, oh_hbm, ow_hbm,
               idxh_v, idxw_v, rhA, rwA, rhB, rwB, sHA, sWA, sHB, sWB):
        wid = lax.axis_index("s") * NC + lax.axis_index("c")
        base0 = wid * EP

        pltpu.sync_copy(hi_hbm.at[pl.ds(base0, EP)], idxh_v)
        pltpu.sync_copy(wi_hbm.at[pl.ds(base0, EP)], idxw_v)

        def start_gA(loc):
            pltpu.async_copy(th_hbm.at[idxh_v.at[pl.ds(loc, CE)]], rhA, sHA)
            pltpu.async_copy(tw_hbm.at[idxw_v.at[pl.ds(loc, CE)]], rwA, sWA)

        def start_gB(loc):
            pltpu.async_copy(th_hbm.at[idxh_v.at[pl.ds(loc, CE)]], rhB, sHB)
            pltpu.async_copy(tw_hbm.at[idxw_v.at[pl.ds(loc, CE)]], rwB, sWB)

        start_gA(0)
        start_gB(CE)

        def body(k, carry):
            locA = pl.multiple_of(2 * k * CE, 8)
            locB = locA + CE
            pltpu.make_async_copy(th_hbm.at[idxh_v.at[pl.ds(locA, CE)]],
                                  rhA, sHA).wait()
            pltpu.make_async_copy(tw_hbm.at[idxw_v.at[pl.ds(locA, CE)]],
                                  rwA, sWA).wait()
            w1 = pltpu.async_copy(rhA, oh_hbm.at[pl.ds(base0 + locA, CE)], sHA)
            w2 = pltpu.async_copy(rwA, ow_hbm.at[pl.ds(base0 + locA, CE)], sWA)
            pltpu.make_async_copy(th_hbm.at[idxh_v.at[pl.ds(locB, CE)]],
                                  rhB, sHB).wait()
            pltpu.make_async_copy(tw_hbm.at[idxw_v.at[pl.ds(locB, CE)]],
                                  rwB, sWB).wait()
            w3 = pltpu.async_copy(rhB, oh_hbm.at[pl.ds(base0 + locB, CE)], sHB)
            w4 = pltpu.async_copy(rwB, ow_hbm.at[pl.ds(base0 + locB, CE)], sWB)
            w1.wait()
            w2.wait()

            @pl.when(locA + 2 * CE < EP)
            def _():
                start_gA(pl.multiple_of(locA + 2 * CE, 8))

            w3.wait()
            w4.wait()

            @pl.when(locB + 2 * CE < EP)
            def _():
                start_gB(pl.multiple_of(locB + 2 * CE, 8))

            return carry

        lax.fori_loop(0, ITERS // 2, body, 0)

    return gather


def _make_scatter(N, E, D):
    info = plsc.get_sparse_core_info()
    NC, NS = info.num_cores, info.num_subcores
    assert NC == 2
    N2 = N // NC
    NR = N2 + 8
    EP = E // NS
    CS = 200
    ITERS = EP // CS
    RT = (NR // NS) // 8 * 8
    TAIL = NR - NS * RT
    assert EP % CS == 0 and CS % 8 == 0 and TAIL % 8 == 0 and N % (2 * 8) == 0

    mesh = plsc.VectorSubcoreMesh(core_axis_name="c", subcore_axis_name="s")

    assert ITERS % 2 == 0

    @functools.partial(
        pl.kernel, mesh=mesh,
        out_type=jax.ShapeDtypeStruct((NC * NR, D), jnp.float32),
        scratch_types=[
            pltpu.VMEM((CS,), jnp.int32),
            pltpu.VMEM((CS,), jnp.int32),
            pltpu.VMEM((CS, D), jnp.float32),
            pltpu.VMEM((CS, D), jnp.float32),
            pltpu.VMEM_SHARED((NR, D), jnp.float32),
            pltpu.SemaphoreType.DMA,
            pltpu.SemaphoreType.DMA,
            pltpu.SemaphoreType.DMA,
            pltpu.SemaphoreType.DMA,
        ],
    )
    def scatter(msg_hbm, hic_hbm, zeros_hbm, out_hbm,
                idxA, idxB, bufA, bufB, acc_sh, sIA, sIB, sMA, sMB):
        cid = lax.axis_index("c")
        sid = lax.axis_index("s")
        base0 = sid * EP

        def start_A(loc):
            pltpu.async_copy(hic_hbm.at[pl.ds(cid * E + loc, CS)], idxA, sIA)
            pltpu.async_copy(msg_hbm.at[pl.ds(loc, CS)], bufA, sMA)

        def start_B(loc):
            pltpu.async_copy(hic_hbm.at[pl.ds(cid * E + loc, CS)], idxB, sIB)
            pltpu.async_copy(msg_hbm.at[pl.ds(loc, CS)], bufB, sMB)

        start_A(pl.multiple_of(base0, 8))
        start_B(pl.multiple_of(base0 + CS, 8))

        pltpu.sync_copy(zeros_hbm.at[pl.ds(sid * RT, RT)],
                        acc_sh.at[pl.ds(sid * RT, RT)])
        if TAIL:
            @pl.when(sid == 0)
            def _():
                pltpu.sync_copy(zeros_hbm.at[pl.ds(NS * RT, TAIL)],
                                acc_sh.at[pl.ds(NS * RT, TAIL)])
        plsc.subcore_barrier()

        def body(k, carry):
            locA = pl.multiple_of(base0 + 2 * k * CS, 8)
            locB = locA + CS
            pltpu.make_async_copy(hic_hbm.at[pl.ds(locA, CS)], idxA, sIA).wait()
            pltpu.make_async_copy(msg_hbm.at[pl.ds(locA, CS)], bufA, sMA).wait()
            pltpu.sync_copy(bufA, acc_sh.at[idxA], add=True)

            @pl.when(2 * k + 2 < ITERS)
            def _():
                start_A(pl.multiple_of(locA + 2 * CS, 8))

            pltpu.make_async_copy(hic_hbm.at[pl.ds(locB, CS)], idxB, sIB).wait()
            pltpu.make_async_copy(msg_hbm.at[pl.ds(locB, CS)], bufB, sMB).wait()
            pltpu.sync_copy(bufB, acc_sh.at[idxB], add=True)

            @pl.when(2 * k + 3 < ITERS)
            def _():
                start_B(pl.multiple_of(locB + 2 * CS, 8))

            return carry

        lax.fori_loop(0, ITERS // 2, body, 0)
        plsc.subcore_barrier()
        pltpu.sync_copy(acc_sh.at[pl.ds(sid * RT, RT)],
                        out_hbm.at[pl.ds(cid * NR + sid * RT, RT)])
        if TAIL:
            @pl.when(sid == 0)
            def _():
                pltpu.sync_copy(acc_sh.at[pl.ds(NS * RT, TAIL)],
                                out_hbm.at[pl.ds(cid * NR + NS * RT, TAIL)])

    return scatter


def kernel(agts, agt_ctrs, ctx, ctx_ctrs, hi, wi,
           dist0_W, dist0_b, dist1_W, dist1_g, dist1_b,
           query_W, query_g, query_b,
           ctx0_W, ctx0_g, ctx0_b, ctx1_W,
           agt_W, norm_g, norm_b, lin_W, lin_g, lin_b):
    N, D = agts.shape
    E = hi.shape[0]

    r2 = lambda v: v.reshape(1, D)
    hi = hi.astype(jnp.int32)
    wi = wi.astype(jnp.int32)

    BN = 2000
    grid_n = N // BN
    row_spec = pl.BlockSpec((BN, D), lambda i: (i, 0))
    tab_spec = pl.BlockSpec((BN, D), lambda i: (i, 0))
    ctr_spec = pl.BlockSpec((BN, 2), lambda i: (i, 0))
    full = lambda s: pl.BlockSpec(s, lambda i: tuple(0 for _ in s))
    table_h, table_w, a0 = pl.pallas_call(
        _node_pre_body,
        grid=(grid_n,),
        in_specs=[row_spec, row_spec, ctr_spec, ctr_spec, full((D, D)),
                  full((1, D)), full((1, D)), full((D, D)), full((D, D)),
                  full((D, D)), full((2, D))],
        out_specs=[tab_spec, tab_spec, row_spec],
        out_shape=[jax.ShapeDtypeStruct((N, D), jnp.int32),
                   jax.ShapeDtypeStruct((N, D), jnp.int32),
                   jax.ShapeDtypeStruct((N, D), jnp.float32)],
    )(agts, ctx, agt_ctrs, ctx_ctrs, query_W.T, r2(query_g), r2(query_b),
      ctx0_W[:, D:2 * D].T, ctx0_W[:, 2 * D:].T, agt_W.T, dist0_W.T)

    gh, gw = _make_gather(N, E, D)(table_h, table_w, hi, wi)

    BE = 2000
    grid_e = E // BE
    espec = pl.BlockSpec((BE, D), lambda i: (i, 0))
    mspec = pl.BlockSpec((BE, D), lambda i: (i, 0))
    msg = pl.pallas_call(
        _edge_body,
        grid=(grid_e,),
        in_specs=[espec, espec, full((1, D)), full((D, D)),
                  full((1, D)), full((1, D)), full((D, D)), full((1, D)),
                  full((1, D)), full((D, D))],
        out_specs=mspec,
        out_shape=jax.ShapeDtypeStruct((E, D), jnp.float32),
    )(gh, gw, r2(dist0_b), dist1_W.T.astype(jnp.bfloat16), r2(dist1_g),
      r2(dist1_b), ctx0_W[:, :D].T.astype(jnp.bfloat16), r2(ctx0_g),
      r2(ctx0_b), ctx1_W.T.astype(jnp.bfloat16))

    hi2d = hi.reshape(E // D, D)
    ispec = pl.BlockSpec((E // D, D), lambda: (0, 0))
    hi0, hi1 = pl.pallas_call(
        functools.partial(_remap_body, n2=N // 2),
        in_specs=[ispec],
        out_specs=[ispec, ispec],
        out_shape=[jax.ShapeDtypeStruct((E // D, D), jnp.int32)] * 2,
    )(hi2d)
    hic = jnp.concatenate([hi0.reshape(E), hi1.reshape(E)])

    NR = N // 2 + 8
    zeros = jnp.zeros((NR, D), jnp.float32)
    parts = _make_scatter(N, E, D)(msg, hic, zeros)
    p = jnp.concatenate([parts[:N // 2], parts[NR:NR + N // 2]], axis=0)

    out = pl.pallas_call(
        _final_body,
        grid=(grid_n,),
        in_specs=[row_spec, row_spec, row_spec, full((1, D)),
                  full((1, D)), full((D, D)), full((1, D)), full((1, D))],
        out_specs=row_spec,
        out_shape=jax.ShapeDtypeStruct((N, D), jnp.float32),
    )(a0, p, agts, r2(norm_g), r2(norm_b), lin_W.T, r2(lin_g), r2(lin_b))
    return out


# speedup vs baseline: 1.2081x; 1.1719x over previous
import functools

import jax
import jax.numpy as jnp
from jax import lax
from jax.experimental import pallas as pl
from jax.experimental.pallas import tpu as pltpu
from jax.experimental.pallas import tpu_sc as plsc

_EPS = 1e-5


def _gn(x, g, b):
    m = jnp.mean(x, axis=-1, keepdims=True)
    v = jnp.mean((x - m) ** 2, axis=-1, keepdims=True)
    return (x - m) * jax.lax.rsqrt(v + _EPS) * g + b


def _node_pre_body(agts_ref, ctx_ref, actr_ref, cctr_ref, qWT_ref, qg_ref,
                   qb_ref, WqT_ref, WcT_ref, aWT_ref, d0WT_ref,
                   th_ref, tw_ref, a0_ref):
    agts = agts_ref[...]
    ctx = ctx_ref[...]
    q = jax.nn.relu(_gn(jnp.dot(agts, qWT_ref[...],
                                preferred_element_type=jnp.float32),
                        qg_ref[...], qb_ref[...]))

    def pack(feat, proj):
        fb = jax.lax.bitcast_convert_type(
            feat.astype(jnp.bfloat16).astype(jnp.float32), jnp.uint32)
        pb = jax.lax.bitcast_convert_type(
            proj.astype(jnp.bfloat16).astype(jnp.float32), jnp.uint32)
        return jax.lax.bitcast_convert_type(
            pb | jax.lax.shift_right_logical(fb, jnp.uint32(16)), jnp.int32)

    th_ref[...] = pack(
        jnp.dot(q, WqT_ref[...], preferred_element_type=jnp.float32),
        jnp.dot(actr_ref[...], d0WT_ref[...],
                preferred_element_type=jnp.float32))
    tw_ref[...] = pack(
        jnp.dot(ctx, WcT_ref[...], preferred_element_type=jnp.float32),
        jnp.dot(cctr_ref[...], d0WT_ref[...],
                preferred_element_type=jnp.float32))
    a0_ref[...] = jnp.dot(agts, aWT_ref[...], preferred_element_type=jnp.float32)


def _edge_body(gh_ref, gw_ref, d0b_ref, d1WT_ref, d1g_ref, d1b_ref,
               WdT_ref, c0g_ref, c0b_ref, c1WT_ref, msg_ref):
    gh = gh_ref[...]
    gw = gw_ref[...]
    _f32 = lambda x: jax.lax.bitcast_convert_type(x, jnp.float32)
    feat = lambda g: _f32(jax.lax.shift_left(g, 16))
    proj = lambda g: _f32(g & jnp.int32(-65536))
    bf = lambda x: x.astype(jnp.bfloat16)
    d0 = jax.nn.relu(proj(gh) - proj(gw) + d0b_ref[...])
    d1 = jax.nn.relu(_gn(jnp.dot(bf(d0), d1WT_ref[...],
                                 preferred_element_type=jnp.float32),
                         d1g_ref[...], d1b_ref[...]))
    pre = (jnp.dot(bf(d1), WdT_ref[...], preferred_element_type=jnp.float32)
           + feat(gh) + feat(gw))
    h = jax.nn.relu(_gn(pre, c0g_ref[...], c0b_ref[...]))
    msg_ref[...] = jnp.dot(bf(h), c1WT_ref[...],
                           preferred_element_type=jnp.float32)


def _remap_body(hi_ref, hi0_ref, hi1_ref, *, n2):
    v = hi_ref[...]
    hi0_ref[...] = jnp.where(v < n2, v, n2)
    hi1_ref[...] = jnp.where(v >= n2, v - n2, n2)


def _final_body(a0_ref, p_ref, q_ref, res_ref, ng_ref, nb_ref, lWT_ref,
                lg_ref, lb_ref, out_ref):
    a = a0_ref[...] + p_ref[...] + q_ref[...]
    a = jax.nn.relu(_gn(a, ng_ref[...], nb_ref[...]))
    a = _gn(jnp.dot(a, lWT_ref[...], preferred_element_type=jnp.float32),
            lg_ref[...], lb_ref[...])
    out_ref[...] = jax.nn.relu(a + res_ref[...])


def _make_gather(N, E, D):
    info = plsc.get_sparse_core_info()
    NC, NS = info.num_cores, info.num_subcores
    NW = NC * NS
    EP = E // NW
    CE = 200
    ITERS = EP // CE
    assert EP % CE == 0 and E % NW == 0

    assert ITERS % 2 == 0
    mesh = plsc.VectorSubcoreMesh(core_axis_name="c", subcore_axis_name="s")
    bshape = jax.ShapeDtypeStruct((E, D), jnp.int32)

    @functools.partial(
        pl.kernel, mesh=mesh,
        out_type=[bshape, bshape],
        scratch_types=[
            pltpu.VMEM((EP,), jnp.int32),
            pltpu.VMEM((EP,), jnp.int32),
            pltpu.VMEM((CE, D), jnp.int32),
            pltpu.VMEM((CE, D), jnp.int32),
            pltpu.VMEM((CE, D), jnp.int32),
            pltpu.VMEM((CE, D), jnp.int32),
            pltpu.SemaphoreType.DMA,
            pltpu.SemaphoreType.DMA,
            pltpu.SemaphoreType.DMA,
            pltpu.SemaphoreType.DMA,
        ],
    )
    def gather(th_hbm, tw_hbm, hi_hbm, wi_hbm, oh_hbm, ow_hbm,
               idxh_v, idxw_v, rhA, rwA, rhB, rwB, sHA, sWA, sHB, sWB):
        wid = lax.axis_index("s") * NC + lax.axis_index("c")
        base0 = wid * EP

        pltpu.sync_copy(hi_hbm.at[pl.ds(base0, EP)], idxh_v)
        pltpu.sync_copy(wi_hbm.at[pl.ds(base0, EP)], idxw_v)

        def start_gA(loc):
            pltpu.async_copy(th_hbm.at[idxh_v.at[pl.ds(loc, CE)]], rhA, sHA)
            pltpu.async_copy(tw_hbm.at[idxw_v.at[pl.ds(loc, CE)]], rwA, sWA)

        def start_gB(loc):
            pltpu.async_copy(th_hbm.at[idxh_v.at[pl.ds(loc, CE)]], rhB, sHB)
            pltpu.async_copy(tw_hbm.at[idxw_v.at[pl.ds(loc, CE)]], rwB, sWB)

        start_gA(0)
        start_gB(CE)

        def body(k, carry):
            locA = pl.multiple_of(2 * k * CE, 8)
            locB = locA + CE
            pltpu.make_async_copy(th_hbm.at[idxh_v.at[pl.ds(locA, CE)]],
                                  rhA, sHA).wait()
            pltpu.make_async_copy(tw_hbm.at[idxw_v.at[pl.ds(locA, CE)]],
                                  rwA, sWA).wait()
            w1 = pltpu.async_copy(rhA, oh_hbm.at[pl.ds(base0 + locA, CE)], sHA)
            w2 = pltpu.async_copy(rwA, ow_hbm.at[pl.ds(base0 + locA, CE)], sWA)
            pltpu.make_async_copy(th_hbm.at[idxh_v.at[pl.ds(locB, CE)]],
                                  rhB, sHB).wait()
            pltpu.make_async_copy(tw_hbm.at[idxw_v.at[pl.ds(locB, CE)]],
                                  rwB, sWB).wait()
            w3 = pltpu.async_copy(rhB, oh_hbm.at[pl.ds(base0 + locB, CE)], sHB)
            w4 = pltpu.async_copy(rwB, ow_hbm.at[pl.ds(base0 + locB, CE)], sWB)
            w1.wait()
            w2.wait()

            @pl.when(locA + 2 * CE < EP)
            def _():
                start_gA(pl.multiple_of(locA + 2 * CE, 8))

            w3.wait()
            w4.wait()

            @pl.when(locB + 2 * CE < EP)
            def _():
                start_gB(pl.multiple_of(locB + 2 * CE, 8))

            return carry

        lax.fori_loop(0, ITERS // 2, body, 0)

    return gather


def _make_scatter(N, E, D):
    info = plsc.get_sparse_core_info()
    NC, NS = info.num_cores, info.num_subcores
    assert NC == 2
    N2 = N // NC
    NR = N2 + 8
    EP = E // NS
    CS = 200
    ITERS = EP // CS
    RT = (NR // NS) // 8 * 8
    TAIL = NR - NS * RT
    assert EP % CS == 0 and CS % 8 == 0 and TAIL % 8 == 0 and N % (2 * 8) == 0

    mesh = plsc.VectorSubcoreMesh(core_axis_name="c", subcore_axis_name="s")

    assert ITERS % 2 == 0

    @functools.partial(
        pl.kernel, mesh=mesh,
        out_type=jax.ShapeDtypeStruct((NC * NR, D), jnp.float32),
        scratch_types=[
            pltpu.VMEM((CS,), jnp.int32),
            pltpu.VMEM((CS,), jnp.int32),
            pltpu.VMEM((CS, D), jnp.float32),
            pltpu.VMEM((CS, D), jnp.float32),
            pltpu.VMEM_SHARED((NR, D), jnp.float32),
            pltpu.SemaphoreType.DMA,
            pltpu.SemaphoreType.DMA,
            pltpu.SemaphoreType.DMA,
            pltpu.SemaphoreType.DMA,
        ],
    )
    def scatter(msg_hbm, hic_hbm, zeros_hbm, out_hbm,
                idxA, idxB, bufA, bufB, acc_sh, sIA, sIB, sMA, sMB):
        cid = lax.axis_index("c")
        sid = lax.axis_index("s")
        base0 = sid * EP

        def start_A(loc):
            pltpu.async_copy(hic_hbm.at[pl.ds(cid * E + loc, CS)], idxA, sIA)
            pltpu.async_copy(msg_hbm.at[pl.ds(loc, CS)], bufA, sMA)

        def start_B(loc):
            pltpu.async_copy(hic_hbm.at[pl.ds(cid * E + loc, CS)], idxB, sIB)
            pltpu.async_copy(msg_hbm.at[pl.ds(loc, CS)], bufB, sMB)

        start_A(pl.multiple_of(base0, 8))
        start_B(pl.multiple_of(base0 + CS, 8))

        pltpu.sync_copy(zeros_hbm.at[pl.ds(sid * RT, RT)],
                        acc_sh.at[pl.ds(sid * RT, RT)])
        if TAIL:
            @pl.when(sid == 0)
            def _():
                pltpu.sync_copy(zeros_hbm.at[pl.ds(NS * RT, TAIL)],
                                acc_sh.at[pl.ds(NS * RT, TAIL)])
        plsc.subcore_barrier()

        def body(k, carry):
            locA = pl.multiple_of(base0 + 2 * k * CS, 8)
            locB = locA + CS
            pltpu.make_async_copy(hic_hbm.at[pl.ds(locA, CS)], idxA, sIA).wait()
            pltpu.make_async_copy(msg_hbm.at[pl.ds(locA, CS)], bufA, sMA).wait()
            pltpu.sync_copy(bufA, acc_sh.at[idxA], add=True)

            @pl.when(2 * k + 2 < ITERS)
            def _():
                start_A(pl.multiple_of(locA + 2 * CS, 8))

            pltpu.make_async_copy(hic_hbm.at[pl.ds(locB, CS)], idxB, sIB).wait()
            pltpu.make_async_copy(msg_hbm.at[pl.ds(locB, CS)], bufB, sMB).wait()
            pltpu.sync_copy(bufB, acc_sh.at[idxB], add=True)

            @pl.when(2 * k + 3 < ITERS)
            def _():
                start_B(pl.multiple_of(locB + 2 * CS, 8))

            return carry

        lax.fori_loop(0, ITERS // 2, body, 0)
        plsc.subcore_barrier()
        pltpu.sync_copy(acc_sh.at[pl.ds(sid * RT, RT)],
                        out_hbm.at[pl.ds(cid * NR + sid * RT, RT)])
        if TAIL:
            @pl.when(sid == 0)
            def _():
                pltpu.sync_copy(acc_sh.at[pl.ds(NS * RT, TAIL)],
                                out_hbm.at[pl.ds(cid * NR + NS * RT, TAIL)])

    return scatter


def kernel(agts, agt_ctrs, ctx, ctx_ctrs, hi, wi,
           dist0_W, dist0_b, dist1_W, dist1_g, dist1_b,
           query_W, query_g, query_b,
           ctx0_W, ctx0_g, ctx0_b, ctx1_W,
           agt_W, norm_g, norm_b, lin_W, lin_g, lin_b):
    N, D = agts.shape
    E = hi.shape[0]

    r2 = lambda v: v.reshape(1, D)
    hi = hi.astype(jnp.int32)
    wi = wi.astype(jnp.int32)

    BN = 2000
    grid_n = N // BN
    row_spec = pl.BlockSpec((BN, D), lambda i: (i, 0))
    tab_spec = pl.BlockSpec((BN, D), lambda i: (i, 0))
    ctr_spec = pl.BlockSpec((BN, 2), lambda i: (i, 0))
    full = lambda s: pl.BlockSpec(s, lambda i: tuple(0 for _ in s))
    table_h, table_w, a0 = pl.pallas_call(
        _node_pre_body,
        grid=(grid_n,),
        in_specs=[row_spec, row_spec, ctr_spec, ctr_spec, full((D, D)),
                  full((1, D)), full((1, D)), full((D, D)), full((D, D)),
                  full((D, D)), full((2, D))],
        out_specs=[tab_spec, tab_spec, row_spec],
        out_shape=[jax.ShapeDtypeStruct((N, D), jnp.int32),
                   jax.ShapeDtypeStruct((N, D), jnp.int32),
                   jax.ShapeDtypeStruct((N, D), jnp.float32)],
    )(agts, ctx, agt_ctrs, ctx_ctrs, query_W.T, r2(query_g), r2(query_b),
      ctx0_W[:, D:2 * D].T, ctx0_W[:, 2 * D:].T, agt_W.T, dist0_W.T)

    BE = 2000
    NR = N // 2 + 8
    zeros = jnp.zeros((NR, D), jnp.float32)
    ps = []
    for (e0, e1) in ((0, 192000 * E // 320000), (192000 * E // 320000, E)):
        Es = e1 - e0
        his, wis = hi[e0:e1], wi[e0:e1]

        gh, gw = _make_gather(N, Es, D)(table_h, table_w, his, wis)

        grid_e = Es // BE
        espec = pl.BlockSpec((BE, D), lambda i: (i, 0))
        mspec = pl.BlockSpec((BE, D), lambda i: (i, 0))
        msg = pl.pallas_call(
            _edge_body,
            grid=(grid_e,),
            in_specs=[espec, espec, full((1, D)), full((D, D)),
                      full((1, D)), full((1, D)), full((D, D)), full((1, D)),
                      full((1, D)), full((D, D))],
            out_specs=mspec,
            out_shape=jax.ShapeDtypeStruct((Es, D), jnp.float32),
        )(gh, gw, r2(dist0_b), dist1_W.T.astype(jnp.bfloat16), r2(dist1_g),
          r2(dist1_b), ctx0_W[:, :D].T.astype(jnp.bfloat16), r2(ctx0_g),
          r2(ctx0_b), ctx1_W.T.astype(jnp.bfloat16))

        hi2d = his.reshape(Es // D, D)
        ispec = pl.BlockSpec((Es // D, D), lambda: (0, 0))
        hi0, hi1 = pl.pallas_call(
            functools.partial(_remap_body, n2=N // 2),
            in_specs=[ispec],
            out_specs=[ispec, ispec],
            out_shape=[jax.ShapeDtypeStruct((Es // D, D), jnp.int32)] * 2,
        )(hi2d)
        hic = jnp.concatenate([hi0.reshape(Es), hi1.reshape(Es)])

        parts = _make_scatter(N, Es, D)(msg, hic, zeros)
        ps.append(jnp.concatenate([parts[:N // 2], parts[NR:NR + N // 2]],
                                  axis=0))

    out = pl.pallas_call(
        _final_body,
        grid=(grid_n,),
        in_specs=[row_spec, row_spec, row_spec, row_spec, full((1, D)),
                  full((1, D)), full((D, D)), full((1, D)), full((1, D))],
        out_specs=row_spec,
        out_shape=jax.ShapeDtypeStruct((N, D), jnp.float32),
    )(a0, ps[0], ps[1], agts, r2(norm_g), r2(norm_b), lin_W.T,
      r2(lin_g), r2(lin_b))
    return out


# speedup vs baseline: 1.3147x; 1.0883x over previous
import functools

import jax
import jax.numpy as jnp
from jax import lax
from jax.experimental import pallas as pl
from jax.experimental.pallas import tpu as pltpu
from jax.experimental.pallas import tpu_sc as plsc

_EPS = 1e-5


def _gn(x, g, b):
    m = jnp.mean(x, axis=-1, keepdims=True)
    v = jnp.mean((x - m) ** 2, axis=-1, keepdims=True)
    return (x - m) * jax.lax.rsqrt(v + _EPS) * g + b


def _node_pre_body(agts_ref, ctx_ref, actr_ref, cctr_ref, qWT_ref, qg_ref,
                   qb_ref, WqT_ref, WcT_ref, aWT_ref, d0WT_ref,
                   th_ref, tw_ref, a0_ref):
    agts = agts_ref[...]
    ctx = ctx_ref[...]
    q = jax.nn.relu(_gn(jnp.dot(agts, qWT_ref[...],
                                preferred_element_type=jnp.float32),
                        qg_ref[...], qb_ref[...]))

    def pack(feat, proj):
        fb = jax.lax.bitcast_convert_type(
            feat.astype(jnp.bfloat16).astype(jnp.float32), jnp.uint32)
        pb = jax.lax.bitcast_convert_type(
            proj.astype(jnp.bfloat16).astype(jnp.float32), jnp.uint32)
        return jax.lax.bitcast_convert_type(
            pb | jax.lax.shift_right_logical(fb, jnp.uint32(16)), jnp.int32)

    th_ref[...] = pack(
        jnp.dot(q, WqT_ref[...], preferred_element_type=jnp.float32),
        jnp.dot(actr_ref[...], d0WT_ref[...],
                preferred_element_type=jnp.float32))
    tw_ref[...] = pack(
        jnp.dot(ctx, WcT_ref[...], preferred_element_type=jnp.float32),
        jnp.dot(cctr_ref[...], d0WT_ref[...],
                preferred_element_type=jnp.float32))
    a0_ref[...] = jnp.dot(agts, aWT_ref[...], preferred_element_type=jnp.float32)


def _edge_body(gh_ref, gw_ref, d0b_ref, d1WT_ref, d1g_ref, d1b_ref,
               WdT_ref, c0g_ref, c0b_ref, c1WT_ref, msg_ref):
    gh = gh_ref[...]
    gw = gw_ref[...]
    _f32 = lambda x: jax.lax.bitcast_convert_type(x, jnp.float32)
    feat = lambda g: _f32(jax.lax.shift_left(g, 16))
    proj = lambda g: _f32(g & jnp.int32(-65536))
    bf = lambda x: x.astype(jnp.bfloat16)
    d0 = jax.nn.relu(proj(gh) - proj(gw) + d0b_ref[...])
    d1 = jax.nn.relu(_gn(jnp.dot(bf(d0), d1WT_ref[...],
                                 preferred_element_type=jnp.float32),
                         d1g_ref[...], d1b_ref[...]))
    pre = (jnp.dot(bf(d1), WdT_ref[...], preferred_element_type=jnp.float32)
           + feat(gh) + feat(gw))
    h = jax.nn.relu(_gn(pre, c0g_ref[...], c0b_ref[...]))
    msg_ref[...] = jnp.dot(bf(h), c1WT_ref[...],
                           preferred_element_type=jnp.float32)


def _remap_body(hi_ref, hi0_ref, hi1_ref, *, n2):
    v = hi_ref[...]
    hi0_ref[...] = jnp.where(v < n2, v, n2)
    hi1_ref[...] = jnp.where(v >= n2, v - n2, n2)


def _final_body(a0_ref, p_ref, q_ref, r_ref, res_ref, ng_ref, nb_ref,
                lWT_ref, lg_ref, lb_ref, out_ref):
    a = a0_ref[...] + p_ref[...] + q_ref[...] + r_ref[...]
    a = jax.nn.relu(_gn(a, ng_ref[...], nb_ref[...]))
    a = _gn(jnp.dot(a, lWT_ref[...], preferred_element_type=jnp.float32),
            lg_ref[...], lb_ref[...])
    out_ref[...] = jax.nn.relu(a + res_ref[...])


def _make_gather(N, E, D):
    info = plsc.get_sparse_core_info()
    NC, NS = info.num_cores, info.num_subcores
    NW = NC * NS
    EP = E // NW
    CE = 200
    ITERS = EP // CE
    assert EP % CE == 0 and E % NW == 0

    assert ITERS % 2 == 0
    mesh = plsc.VectorSubcoreMesh(core_axis_name="c", subcore_axis_name="s")
    bshape = jax.ShapeDtypeStruct((E, D), jnp.int32)

    @functools.partial(
        pl.kernel, mesh=mesh,
        out_type=[bshape, bshape],
        scratch_types=[
            pltpu.VMEM((EP,), jnp.int32),
            pltpu.VMEM((EP,), jnp.int32),
            pltpu.VMEM((CE, D), jnp.int32),
            pltpu.VMEM((CE, D), jnp.int32),
            pltpu.VMEM((CE, D), jnp.int32),
            pltpu.VMEM((CE, D), jnp.int32),
            pltpu.SemaphoreType.DMA,
            pltpu.SemaphoreType.DMA,
            pltpu.SemaphoreType.DMA,
            pltpu.SemaphoreType.DMA,
        ],
    )
    def gather(th_hbm, tw_hbm, hi_hbm, wi_hbm, oh_hbm, ow_hbm,
               idxh_v, idxw_v, rhA, rwA, rhB, rwB, sHA, sWA, sHB, sWB):
        wid = lax.axis_index("s") * NC + lax.axis_index("c")
        base0 = wid * EP

        pltpu.sync_copy(hi_hbm.at[pl.ds(base0, EP)], idxh_v)
        pltpu.sync_copy(wi_hbm.at[pl.ds(base0, EP)], idxw_v)

        def start_gA(loc):
            pltpu.async_copy(th_hbm.at[idxh_v.at[pl.ds(loc, CE)]], rhA, sHA)
            pltpu.async_copy(tw_hbm.at[idxw_v.at[pl.ds(loc, CE)]], rwA, sWA)

        def start_gB(loc):
            pltpu.async_copy(th_hbm.at[idxh_v.at[pl.ds(loc, CE)]], rhB, sHB)
            pltpu.async_copy(tw_hbm.at[idxw_v.at[pl.ds(loc, CE)]], rwB, sWB)

        start_gA(0)
        start_gB(CE)

        def body(k, carry):
            locA = pl.multiple_of(2 * k * CE, 8)
            locB = locA + CE
            pltpu.make_async_copy(th_hbm.at[idxh_v.at[pl.ds(locA, CE)]],
                                  rhA, sHA).wait()
            pltpu.make_async_copy(tw_hbm.at[idxw_v.at[pl.ds(locA, CE)]],
                                  rwA, sWA).wait()
            w1 = pltpu.async_copy(rhA, oh_hbm.at[pl.ds(base0 + locA, CE)], sHA)
            w2 = pltpu.async_copy(rwA, ow_hbm.at[pl.ds(base0 + locA, CE)], sWA)
            pltpu.make_async_copy(th_hbm.at[idxh_v.at[pl.ds(locB, CE)]],
                                  rhB, sHB).wait()
            pltpu.make_async_copy(tw_hbm.at[idxw_v.at[pl.ds(locB, CE)]],
                                  rwB, sWB).wait()
            w3 = pltpu.async_copy(rhB, oh_hbm.at[pl.ds(base0 + locB, CE)], sHB)
            w4 = pltpu.async_copy(rwB, ow_hbm.at[pl.ds(base0 + locB, CE)], sWB)
            w1.wait()
            w2.wait()

            @pl.when(locA + 2 * CE < EP)
            def _():
                start_gA(pl.multiple_of(locA + 2 * CE, 8))

            w3.wait()
            w4.wait()

            @pl.when(locB + 2 * CE < EP)
            def _():
                start_gB(pl.multiple_of(locB + 2 * CE, 8))

            return carry

        lax.fori_loop(0, ITERS // 2, body, 0)

    return gather


def _make_scatter(N, E, D):
    info = plsc.get_sparse_core_info()
    NC, NS = info.num_cores, info.num_subcores
    assert NC == 2
    N2 = N // NC
    NR = N2 + 8
    EP = E // NS
    CS = 200
    ITERS = EP // CS
    RT = (NR // NS) // 8 * 8
    TAIL = NR - NS * RT
    assert EP % CS == 0 and CS % 8 == 0 and TAIL % 8 == 0 and N % (2 * 8) == 0

    mesh = plsc.VectorSubcoreMesh(core_axis_name="c", subcore_axis_name="s")

    assert ITERS % 2 == 0

    @functools.partial(
        pl.kernel, mesh=mesh,
        out_type=jax.ShapeDtypeStruct((NC * NR, D), jnp.float32),
        scratch_types=[
            pltpu.VMEM((CS,), jnp.int32),
            pltpu.VMEM((CS,), jnp.int32),
            pltpu.VMEM((CS, D), jnp.float32),
            pltpu.VMEM((CS, D), jnp.float32),
            pltpu.VMEM_SHARED((NR, D), jnp.float32),
            pltpu.SemaphoreType.DMA,
            pltpu.SemaphoreType.DMA,
            pltpu.SemaphoreType.DMA,
            pltpu.SemaphoreType.DMA,
        ],
    )
    def scatter(msg_hbm, hic_hbm, zeros_hbm, out_hbm,
                idxA, idxB, bufA, bufB, acc_sh, sIA, sIB, sMA, sMB):
        cid = lax.axis_index("c")
        sid = lax.axis_index("s")
        base0 = sid * EP

        def start_A(loc):
            pltpu.async_copy(hic_hbm.at[pl.ds(cid * E + loc, CS)], idxA, sIA)
            pltpu.async_copy(msg_hbm.at[pl.ds(loc, CS)], bufA, sMA)

        def start_B(loc):
            pltpu.async_copy(hic_hbm.at[pl.ds(cid * E + loc, CS)], idxB, sIB)
            pltpu.async_copy(msg_hbm.at[pl.ds(loc, CS)], bufB, sMB)

        start_A(pl.multiple_of(base0, 8))
        start_B(pl.multiple_of(base0 + CS, 8))

        pltpu.sync_copy(zeros_hbm.at[pl.ds(sid * RT, RT)],
                        acc_sh.at[pl.ds(sid * RT, RT)])
        if TAIL:
            @pl.when(sid == 0)
            def _():
                pltpu.sync_copy(zeros_hbm.at[pl.ds(NS * RT, TAIL)],
                                acc_sh.at[pl.ds(NS * RT, TAIL)])
        plsc.subcore_barrier()

        def body(k, carry):
            locA = pl.multiple_of(base0 + 2 * k * CS, 8)
            locB = locA + CS
            pltpu.make_async_copy(hic_hbm.at[pl.ds(locA, CS)], idxA, sIA).wait()
            pltpu.make_async_copy(msg_hbm.at[pl.ds(locA, CS)], bufA, sMA).wait()
            pltpu.sync_copy(bufA, acc_sh.at[idxA], add=True)

            @pl.when(2 * k + 2 < ITERS)
            def _():
                start_A(pl.multiple_of(locA + 2 * CS, 8))

            pltpu.make_async_copy(hic_hbm.at[pl.ds(locB, CS)], idxB, sIB).wait()
            pltpu.make_async_copy(msg_hbm.at[pl.ds(locB, CS)], bufB, sMB).wait()
            pltpu.sync_copy(bufB, acc_sh.at[idxB], add=True)

            @pl.when(2 * k + 3 < ITERS)
            def _():
                start_B(pl.multiple_of(locB + 2 * CS, 8))

            return carry

        lax.fori_loop(0, ITERS // 2, body, 0)
        plsc.subcore_barrier()
        pltpu.sync_copy(acc_sh.at[pl.ds(sid * RT, RT)],
                        out_hbm.at[pl.ds(cid * NR + sid * RT, RT)])
        if TAIL:
            @pl.when(sid == 0)
            def _():
                pltpu.sync_copy(acc_sh.at[pl.ds(NS * RT, TAIL)],
                                out_hbm.at[pl.ds(cid * NR + NS * RT, TAIL)])

    return scatter


def kernel(agts, agt_ctrs, ctx, ctx_ctrs, hi, wi,
           dist0_W, dist0_b, dist1_W, dist1_g, dist1_b,
           query_W, query_g, query_b,
           ctx0_W, ctx0_g, ctx0_b, ctx1_W,
           agt_W, norm_g, norm_b, lin_W, lin_g, lin_b):
    N, D = agts.shape
    E = hi.shape[0]

    r2 = lambda v: v.reshape(1, D)
    hi = hi.astype(jnp.int32)
    wi = wi.astype(jnp.int32)

    BN = 2000
    grid_n = N // BN
    row_spec = pl.BlockSpec((BN, D), lambda i: (i, 0))
    tab_spec = pl.BlockSpec((BN, D), lambda i: (i, 0))
    ctr_spec = pl.BlockSpec((BN, 2), lambda i: (i, 0))
    full = lambda s: pl.BlockSpec(s, lambda i: tuple(0 for _ in s))
    table_h, table_w, a0 = pl.pallas_call(
        _node_pre_body,
        grid=(grid_n,),
        in_specs=[row_spec, row_spec, ctr_spec, ctr_spec, full((D, D)),
                  full((1, D)), full((1, D)), full((D, D)), full((D, D)),
                  full((D, D)), full((2, D))],
        out_specs=[tab_spec, tab_spec, row_spec],
        out_shape=[jax.ShapeDtypeStruct((N, D), jnp.int32),
                   jax.ShapeDtypeStruct((N, D), jnp.int32),
                   jax.ShapeDtypeStruct((N, D), jnp.float32)],
    )(agts, ctx, agt_ctrs, ctx_ctrs, query_W.T, r2(query_g), r2(query_b),
      ctx0_W[:, D:2 * D].T, ctx0_W[:, 2 * D:].T, agt_W.T, dist0_W.T)

    BE = 2000
    NR = N // 2 + 8
    zeros = jnp.zeros((NR, D), jnp.float32)
    ps = []
    cut1, cut2 = 64000 * E // 320000, 192000 * E // 320000
    for (e0, e1) in ((0, cut1), (cut1, cut2), (cut2, E)):
        Es = e1 - e0
        his, wis = hi[e0:e1], wi[e0:e1]

        gh, gw = _make_gather(N, Es, D)(table_h, table_w, his, wis)

        grid_e = Es // BE
        espec = pl.BlockSpec((BE, D), lambda i: (i, 0))
        mspec = pl.BlockSpec((BE, D), lambda i: (i, 0))
        msg = pl.pallas_call(
            _edge_body,
            grid=(grid_e,),
            in_specs=[espec, espec, full((1, D)), full((D, D)),
                      full((1, D)), full((1, D)), full((D, D)), full((1, D)),
                      full((1, D)), full((D, D))],
            out_specs=mspec,
            out_shape=jax.ShapeDtypeStruct((Es, D), jnp.float32),
        )(gh, gw, r2(dist0_b), dist1_W.T.astype(jnp.bfloat16), r2(dist1_g),
          r2(dist1_b), ctx0_W[:, :D].T.astype(jnp.bfloat16), r2(ctx0_g),
          r2(ctx0_b), ctx1_W.T.astype(jnp.bfloat16))

        hi2d = his.reshape(Es // D, D)
        ispec = pl.BlockSpec((Es // D, D), lambda: (0, 0))
        hi0, hi1 = pl.pallas_call(
            functools.partial(_remap_body, n2=N // 2),
            in_specs=[ispec],
            out_specs=[ispec, ispec],
            out_shape=[jax.ShapeDtypeStruct((Es // D, D), jnp.int32)] * 2,
        )(hi2d)
        hic = jnp.concatenate([hi0.reshape(Es), hi1.reshape(Es)])

        parts = _make_scatter(N, Es, D)(msg, hic, zeros)
        ps.append(jnp.concatenate([parts[:N // 2], parts[NR:NR + N // 2]],
                                  axis=0))

    out = pl.pallas_call(
        _final_body,
        grid=(grid_n,),
        in_specs=[row_spec, row_spec, row_spec, row_spec, row_spec,
                  full((1, D)), full((1, D)), full((D, D)), full((1, D)),
                  full((1, D))],
        out_specs=row_spec,
        out_shape=jax.ShapeDtypeStruct((N, D), jnp.float32),
    )(a0, ps[0], ps[1], ps[2], agts, r2(norm_g), r2(norm_b), lin_W.T,
      r2(lin_g), r2(lin_b))
    return out
